# Initial kernel scaffold; baseline (speedup 1.0000x reference)
import jax
import jax.numpy as jnp
from jax.experimental import pallas as pl


def kernel(bag_encoding, batch_indices, Vw, Vb, ww, wb, Dw, Db):
    raise NotImplementedError("write your pallas kernel here")


# speedup vs baseline: 2.4079x; 2.4079x over previous
import functools

import jax
import jax.numpy as jnp
from jax import lax
from jax.experimental import pallas as pl
from jax.experimental.pallas import tpu as pltpu
from jax.experimental.pallas import tpu_sc as plsc

N = 320000
D = 128
H = 64
NUM_BAGS = 10000

NC = 1
NS = 16
NW = NC * NS
ROWS_PER_W = N // NW
CHUNK = 160
NCHUNK = ROWS_PER_W // CHUNK
SCAT = 80
NSCAT = CHUNK // SCAT
STRIPE = 1000
NSTRIPERS = NUM_BAGS // STRIPE

BT = 512


def _alpha_body(x_ref, vw_ref, aux_ref, o_ref):
    x = x_ref[...]
    vw = vw_ref[...]
    v = jnp.tanh(
        lax.dot_general(x, vw, (((1,), (1,)), ((), ())),
                        preferred_element_type=jnp.float32)
        + aux_ref[0:1, :])
    s = jnp.sum(v * aux_ref[1:2, :], axis=1)
    o_ref[...] = jnp.exp(s)


def _alpha_pass(x, vw, aux):
    grid = N // BT
    return pl.pallas_call(
        _alpha_body,
        grid=(grid,),
        in_specs=[
            pl.BlockSpec((BT, D), lambda i: (i, 0)),
            pl.BlockSpec((H, D), lambda i: (0, 0)),
            pl.BlockSpec((8, H), lambda i: (0, 0)),
        ],
        out_specs=pl.BlockSpec((BT,), lambda i: (i,)),
        out_shape=jax.ShapeDtypeStruct((N,), jnp.float32),
    )(x, vw, aux)


def _segsum_body(x_hbm, alpha_hbm, idx_hbm, outx_hbm, outa_hbm,
                 rows_v, a16_v, alpha_v, idx_v, accx_s, acca_s):
    c = lax.axis_index("c")
    s = lax.axis_index("s")
    w = c * NS + s

    def _zrow(r, carry):
        for j in range(D // 16):
            rows_v[r, pl.ds(16 * j, 16)] = jnp.zeros((16,), jnp.float32)
        a16_v[r, :] = jnp.zeros((16,), jnp.float32)
        return carry

    lax.fori_loop(0, CHUNK, _zrow, 0, unroll=1)
    zbase = s * STRIPE

    @pl.when(s < NSTRIPERS)
    def _zero_stripe():
        for off in range(0, STRIPE, CHUNK):
            ln = min(CHUNK, STRIPE - off)
            pltpu.sync_copy(rows_v.at[pl.ds(0, ln)],
                            accx_s.at[pl.ds(zbase + off, ln)])
            pltpu.sync_copy(a16_v.at[pl.ds(0, ln)],
                            acca_s.at[pl.ds(zbase + off, ln)])

    plsc.subcore_barrier()

    def _chunk(t, carry):
        base = pl.multiple_of(w * ROWS_PER_W + t * CHUNK, 8)
        pltpu.sync_copy(x_hbm.at[pl.ds(base, CHUNK)], rows_v)
        pltpu.sync_copy(alpha_hbm.at[pl.ds(base, CHUNK)], alpha_v)
        for j in range(NSCAT):
            pltpu.sync_copy(idx_hbm.at[pl.ds(base + SCAT * j, SCAT)],
                            idx_v.at[j])

        def _rowgrp(g, carry2):
            av = alpha_v[pl.ds(16 * g, 16)]
            for k in range(16):
                r = 16 * g + k
                a = jnp.full((16,), av[k], jnp.float32)
                for j in range(D // 16):
                    rows_v[r, pl.ds(16 * j, 16)] = (
                        rows_v[r, pl.ds(16 * j, 16)] * a)
                a16_v[r, :] = a
            return carry2

        lax.fori_loop(0, CHUNK // 16, _rowgrp, 0, unroll=1)

        for j in range(NSCAT):
            pltpu.sync_copy(rows_v.at[pl.ds(SCAT * j, SCAT)],
                            accx_s.at[idx_v.at[j]], add=True)
            pltpu.sync_copy(a16_v.at[pl.ds(SCAT * j, SCAT)],
                            acca_s.at[idx_v.at[j]], add=True)
        return carry

    lax.fori_loop(0, NCHUNK, _chunk, 0, unroll=1)

    plsc.subcore_barrier()

    @pl.when(s < NSTRIPERS)
    def _copy_out():
        pltpu.sync_copy(accx_s.at[pl.ds(zbase, STRIPE)],
                        outx_hbm.at[pl.ds(zbase, STRIPE)])
        pltpu.sync_copy(acca_s.at[pl.ds(zbase, STRIPE)],
                        outa_hbm.at[pl.ds(zbase, STRIPE)])


def _segsum_pass(x, alpha, idx):
    mesh = plsc.VectorSubcoreMesh(core_axis_name="c", subcore_axis_name="s",
                                  num_cores=NC)
    fn = pl.kernel(
        _segsum_body,
        out_type=[
            jax.ShapeDtypeStruct((NUM_BAGS, D), jnp.float32),
            jax.ShapeDtypeStruct((NUM_BAGS, 16), jnp.float32),
        ],
        mesh=mesh,
        scratch_types=[
            pltpu.VMEM((CHUNK, D), jnp.float32),
            pltpu.VMEM((CHUNK, 16), jnp.float32),
            pltpu.VMEM((CHUNK,), jnp.float32),
            pltpu.VMEM((NSCAT, SCAT), jnp.int32),
            pltpu.VMEM_SHARED((NUM_BAGS, D), jnp.float32),
            pltpu.VMEM_SHARED((NUM_BAGS, 16), jnp.float32),
        ],
        compiler_params=pltpu.CompilerParams(use_tc_tiling_on_sc=False),
    )
    return fn(x, alpha, idx)


def _finish_body(sx_ref, sa_ref, daux_ref, o_ref):
    ssum = sx_ref[...]
    asum = sa_ref[:, 0:1]
    safe = jnp.where(asum != 0.0, asum, 1.0)
    bag = jnp.where(asum != 0.0, ssum / safe, 0.0)
    dw = daux_ref[0:2, :]
    logits = lax.dot_general(bag, dw, (((1,), (1,)), ((), ())),
                             preferred_element_type=jnp.float32)
    logits = logits + daux_ref[2:3, 0:2]
    m = jnp.max(logits, axis=1, keepdims=True)
    e = jnp.exp(logits - m)
    o_ref[...] = e / jnp.sum(e, axis=1, keepdims=True)


def _finish_pass(sx, sa, daux):
    return pl.pallas_call(
        _finish_body,
        out_shape=jax.ShapeDtypeStruct((NUM_BAGS, 2), jnp.float32),
    )(sx, sa, daux)


def kernel(bag_encoding, batch_indices, Vw, Vb, ww, wb, Dw, Db):
    x = bag_encoding
    aux = jnp.zeros((8, H), jnp.float32).at[0].set(Vb).at[1].set(ww[0])
    alpha = _alpha_pass(x, Vw, aux)
    sx, sa = _segsum_pass(x, alpha, batch_indices.astype(jnp.int32))
    daux = jnp.zeros((8, D), jnp.float32).at[0:2].set(Dw).at[2, 0:2].set(Db)
    return _finish_pass(sx, sa, daux)


# speedup vs baseline: 3.3282x; 1.3822x over previous
import functools

import jax
import jax.numpy as jnp
from jax import lax
from jax.experimental import pallas as pl
from jax.experimental.pallas import tpu as pltpu
from jax.experimental.pallas import tpu_sc as plsc

N = 320000
D = 128
H = 64
NUM_BAGS = 10000

NC = 1
NS = 16
NW = NC * NS
ROWS_PER_W = N // NW
CHUNK = 80
NCHUNK = ROWS_PER_W // CHUNK
STRIPE = 1000
NSTRIPERS = NUM_BAGS // STRIPE

BT = 512


def _alpha_body(x_ref, vw_ref, aux_ref, o_ref):
    x = x_ref[...]
    vw = vw_ref[...]
    v = jnp.tanh(
        lax.dot_general(x, vw, (((1,), (1,)), ((), ())),
                        preferred_element_type=jnp.float32)
        + aux_ref[0:1, :])
    s = jnp.sum(v * aux_ref[1:2, :], axis=1)
    o_ref[...] = jnp.exp(s)


def _alpha_pass(x, vw, aux):
    grid = N // BT
    return pl.pallas_call(
        _alpha_body,
        grid=(grid,),
        in_specs=[
            pl.BlockSpec((BT, D), lambda i: (i, 0)),
            pl.BlockSpec((H, D), lambda i: (0, 0)),
            pl.BlockSpec((8, H), lambda i: (0, 0)),
        ],
        out_specs=pl.BlockSpec((BT,), lambda i: (i,)),
        out_shape=jax.ShapeDtypeStruct((N,), jnp.float32),
    )(x, vw, aux)


def _segsum_body(x_hbm, alpha_hbm, idx_hbm, outx_hbm, outa_hbm,
                 rows_v0, rows_v1, a16_v0, a16_v1, alpha_v0, alpha_v1,
                 idx_v0, idx_v1, sem0, sem1, accx_s, acca_s):
    c = lax.axis_index("c")
    s = lax.axis_index("s")
    w = c * NS + s
    rows_b = (rows_v0, rows_v1)
    a16_b = (a16_v0, a16_v1)
    alpha_b = (alpha_v0, alpha_v1)
    idx_b = (idx_v0, idx_v1)
    sem_b = (sem0, sem1)

    def _zrow(r, carry):
        for j in range(D // 16):
            rows_v0[r, pl.ds(16 * j, 16)] = jnp.zeros((16,), jnp.float32)
        a16_v0[r, :] = jnp.zeros((16,), jnp.float32)
        return carry

    lax.fori_loop(0, CHUNK, _zrow, 0, unroll=1)
    zbase = s * STRIPE

    @pl.when(s < NSTRIPERS)
    def _zero_stripe():
        for off in range(0, STRIPE, CHUNK):
            ln = min(CHUNK, STRIPE - off)
            pltpu.sync_copy(rows_v0.at[pl.ds(0, ln)],
                            accx_s.at[pl.ds(zbase + off, ln)])
            pltpu.sync_copy(a16_v0.at[pl.ds(0, ln)],
                            acca_s.at[pl.ds(zbase + off, ln)])

    plsc.subcore_barrier()

    def _start_loads(t, b):
        base = pl.multiple_of(w * ROWS_PER_W + t * CHUNK, 8)
        pltpu.async_copy(x_hbm.at[pl.ds(base, CHUNK)], rows_b[b], sem_b[b])
        pltpu.async_copy(alpha_hbm.at[pl.ds(base, CHUNK)], alpha_b[b],
                         sem_b[b])
        pltpu.async_copy(idx_hbm.at[pl.ds(base, CHUNK)], idx_b[b].at[0],
                         sem_b[b])

    def _wait_loads(b):
        pltpu.make_async_copy(x_hbm.at[pl.ds(0, CHUNK)], rows_b[b],
                              sem_b[b]).wait()
        pltpu.make_async_copy(alpha_hbm.at[pl.ds(0, CHUNK)], alpha_b[b],
                              sem_b[b]).wait()
        pltpu.make_async_copy(idx_hbm.at[pl.ds(0, CHUNK)], idx_b[b].at[0],
                              sem_b[b]).wait()

    def _process(b):
        rows_v, a16_v, alpha_v, idx_v = (rows_b[b], a16_b[b], alpha_b[b],
                                         idx_b[b])

        def _rowgrp(g, carry2):
            av = alpha_v[pl.ds(16 * g, 16)]
            for k in range(16):
                r = 16 * g + k
                a = jnp.full((16,), av[k], jnp.float32)
                for j in range(D // 16):
                    rows_v[r, pl.ds(16 * j, 16)] = (
                        rows_v[r, pl.ds(16 * j, 16)] * a)
                a16_v[r, :] = a
            return carry2

        lax.fori_loop(0, CHUNK // 16, _rowgrp, 0, unroll=1)
        pltpu.sync_copy(rows_v, accx_s.at[idx_v.at[0]], add=True)
        pltpu.sync_copy(a16_v, acca_s.at[idx_v.at[0]], add=True)

    _start_loads(0, 0)
    last = NCHUNK - 1

    def _pair(u, carry):
        t0 = 2 * u
        _wait_loads(0)
        _start_loads(jnp.minimum(t0 + 1, last), 1)
        _process(0)
        _wait_loads(1)
        _start_loads(jnp.minimum(t0 + 2, last), 0)
        _process(1)
        return carry

    lax.fori_loop(0, NCHUNK // 2, _pair, 0, unroll=1)
    _wait_loads(0)

    plsc.subcore_barrier()

    @pl.when(s < NSTRIPERS)
    def _copy_out():
        pltpu.sync_copy(accx_s.at[pl.ds(zbase, STRIPE)],
                        outx_hbm.at[pl.ds(zbase, STRIPE)])
        pltpu.sync_copy(acca_s.at[pl.ds(zbase, STRIPE)],
                        outa_hbm.at[pl.ds(zbase, STRIPE)])


def _segsum_pass(x, alpha, idx):
    mesh = plsc.VectorSubcoreMesh(core_axis_name="c", subcore_axis_name="s",
                                  num_cores=NC)
    fn = pl.kernel(
        _segsum_body,
        out_type=[
            jax.ShapeDtypeStruct((NUM_BAGS, D), jnp.float32),
            jax.ShapeDtypeStruct((NUM_BAGS, 16), jnp.float32),
        ],
        mesh=mesh,
        scratch_types=[
            pltpu.VMEM((CHUNK, D), jnp.float32),
            pltpu.VMEM((CHUNK, D), jnp.float32),
            pltpu.VMEM((CHUNK, 16), jnp.float32),
            pltpu.VMEM((CHUNK, 16), jnp.float32),
            pltpu.VMEM((CHUNK,), jnp.float32),
            pltpu.VMEM((CHUNK,), jnp.float32),
            pltpu.VMEM((1, CHUNK), jnp.int32),
            pltpu.VMEM((1, CHUNK), jnp.int32),
            pltpu.SemaphoreType.DMA,
            pltpu.SemaphoreType.DMA,
            pltpu.VMEM_SHARED((NUM_BAGS, D), jnp.float32),
            pltpu.VMEM_SHARED((NUM_BAGS, 16), jnp.float32),
        ],
        compiler_params=pltpu.CompilerParams(use_tc_tiling_on_sc=False),
    )
    return fn(x, alpha, idx)


def _finish_body(sx_ref, sa_ref, daux_ref, o_ref):
    ssum = sx_ref[...]
    asum = sa_ref[:, 0:1]
    safe = jnp.where(asum != 0.0, asum, 1.0)
    bag = jnp.where(asum != 0.0, ssum / safe, 0.0)
    dw = daux_ref[0:2, :]
    logits = lax.dot_general(bag, dw, (((1,), (1,)), ((), ())),
                             preferred_element_type=jnp.float32)
    logits = logits + daux_ref[2:3, 0:2]
    m = jnp.max(logits, axis=1, keepdims=True)
    e = jnp.exp(logits - m)
    o_ref[...] = e / jnp.sum(e, axis=1, keepdims=True)


def _finish_pass(sx, sa, daux):
    return pl.pallas_call(
        _finish_body,
        out_shape=jax.ShapeDtypeStruct((NUM_BAGS, 2), jnp.float32),
    )(sx, sa, daux)


def kernel(bag_encoding, batch_indices, Vw, Vb, ww, wb, Dw, Db):
    x = bag_encoding
    aux = jnp.zeros((8, H), jnp.float32).at[0].set(Vb).at[1].set(ww[0])
    alpha = _alpha_pass(x, Vw, aux)
    sx, sa = _segsum_pass(x, alpha, batch_indices.astype(jnp.int32))
    daux = jnp.zeros((8, D), jnp.float32).at[0:2].set(Dw).at[2, 0:2].set(Db)
    return _finish_pass(sx, sa, daux)


# speedup vs baseline: 5.5942x; 1.6808x over previous
import functools

import jax
import jax.numpy as jnp
from jax import lax
from jax.experimental import pallas as pl
from jax.experimental.pallas import tpu as pltpu
from jax.experimental.pallas import tpu_sc as plsc

N = 320000
D = 128
H = 64
NUM_BAGS = 10000

NC = 1
NS = 16
NW = NC * NS
ROWS_PER_W = N // NW
CHUNK = 80
NCHUNK = ROWS_PER_W // CHUNK
STRIPE = 1000
NSTRIPERS = NUM_BAGS // STRIPE

BT = 2048


def _alpha_body(x_ref, vw_ref, vbww_ref, o_ref):
    x = x_ref[...]
    vw = vw_ref[...]
    vt = jnp.tanh(
        lax.dot_general(vw, x, (((1,), (1,)), ((), ())),
                        preferred_element_type=jnp.float32)
        + vbww_ref[:, 0:1])
    s = jnp.sum(vt * vbww_ref[:, 1:2], axis=0)
    o_ref[...] = jnp.exp(s)


def _alpha_pass(x, vw, aux):
    grid = pl.cdiv(N, BT)
    return pl.pallas_call(
        _alpha_body,
        grid=(grid,),
        in_specs=[
            pl.BlockSpec((BT, D), lambda i: (i, 0)),
            pl.BlockSpec((H, D), lambda i: (0, 0)),
            pl.BlockSpec((H, 8), lambda i: (0, 0)),
        ],
        out_specs=pl.BlockSpec((BT,), lambda i: (i,)),
        out_shape=jax.ShapeDtypeStruct((N,), jnp.float32),
    )(x, vw, aux)


def _segsum_body(x_hbm, alpha_hbm, idx_hbm, outx_hbm, outa_hbm,
                 rows_v0, rows_v1, a16_v0, a16_v1, alpha_v0, alpha_v1,
                 idx_v0, idx_v1, sem0, sem1, accx_s, acca_s):
    c = lax.axis_index("c")
    s = lax.axis_index("s")
    w = c * NS + s
    rows_b = (rows_v0, rows_v1)
    a16_b = (a16_v0, a16_v1)
    alpha_b = (alpha_v0, alpha_v1)
    idx_b = (idx_v0, idx_v1)
    sem_b = (sem0, sem1)

    def _zrow(r, carry):
        for j in range(D // 16):
            rows_v0[r, pl.ds(16 * j, 16)] = jnp.zeros((16,), jnp.float32)
        a16_v0[r, :] = jnp.zeros((16,), jnp.float32)
        return carry

    lax.fori_loop(0, CHUNK, _zrow, 0, unroll=1)
    zbase = s * STRIPE

    @pl.when(s < NSTRIPERS)
    def _zero_stripe():
        for off in range(0, STRIPE, CHUNK):
            ln = min(CHUNK, STRIPE - off)
            pltpu.sync_copy(rows_v0.at[pl.ds(0, ln)],
                            accx_s.at[pl.ds(zbase + off, ln)])
            pltpu.sync_copy(a16_v0.at[pl.ds(0, ln)],
                            acca_s.at[pl.ds(zbase + off, ln)])

    plsc.subcore_barrier()

    def _start_loads(t, b):
        base = pl.multiple_of(w * ROWS_PER_W + t * CHUNK, 8)
        pltpu.async_copy(x_hbm.at[pl.ds(base, CHUNK)], rows_b[b], sem_b[b])
        pltpu.async_copy(alpha_hbm.at[pl.ds(base, CHUNK)], alpha_b[b],
                         sem_b[b])
        pltpu.async_copy(idx_hbm.at[pl.ds(base, CHUNK)], idx_b[b].at[0],
                         sem_b[b])

    def _wait_loads(b):
        pltpu.make_async_copy(x_hbm.at[pl.ds(0, CHUNK)], rows_b[b],
                              sem_b[b]).wait()
        pltpu.make_async_copy(alpha_hbm.at[pl.ds(0, CHUNK)], alpha_b[b],
                              sem_b[b]).wait()
        pltpu.make_async_copy(idx_hbm.at[pl.ds(0, CHUNK)], idx_b[b].at[0],
                              sem_b[b]).wait()

    def _process(b):
        rows_v, a16_v, alpha_v, idx_v = (rows_b[b], a16_b[b], alpha_b[b],
                                         idx_b[b])

        def _rowgrp(g, carry2):
            av = alpha_v[pl.ds(16 * g, 16)]
            for k in range(16):
                r = 16 * g + k
                a = jnp.full((16,), av[k], jnp.float32)
                for j in range(D // 16):
                    rows_v[r, pl.ds(16 * j, 16)] = (
                        rows_v[r, pl.ds(16 * j, 16)] * a)
                a16_v[r, :] = a
            return carry2

        lax.fori_loop(0, CHUNK // 16, _rowgrp, 0, unroll=1)
        pltpu.sync_copy(rows_v, accx_s.at[idx_v.at[0]], add=True)
        pltpu.sync_copy(a16_v, acca_s.at[idx_v.at[0]], add=True)

    _start_loads(0, 0)
    last = NCHUNK - 1

    def _pair(u, carry):
        t0 = 2 * u
        _wait_loads(0)
        _start_loads(jnp.minimum(t0 + 1, last), 1)
        _process(0)
        _wait_loads(1)
        _start_loads(jnp.minimum(t0 + 2, last), 0)
        _process(1)
        return carry

    lax.fori_loop(0, NCHUNK // 2, _pair, 0, unroll=1)
    _wait_loads(0)

    plsc.subcore_barrier()

    @pl.when(s < NSTRIPERS)
    def _copy_out():
        pltpu.sync_copy(accx_s.at[pl.ds(zbase, STRIPE)],
                        outx_hbm.at[pl.ds(zbase, STRIPE)])
        pltpu.sync_copy(acca_s.at[pl.ds(zbase, STRIPE)],
                        outa_hbm.at[pl.ds(zbase, STRIPE)])


def _segsum_pass(x, alpha, idx):
    mesh = plsc.VectorSubcoreMesh(core_axis_name="c", subcore_axis_name="s",
                                  num_cores=NC)
    fn = pl.kernel(
        _segsum_body,
        out_type=[
            jax.ShapeDtypeStruct((NUM_BAGS, D), jnp.float32),
            jax.ShapeDtypeStruct((NUM_BAGS, 16), jnp.float32),
        ],
        mesh=mesh,
        scratch_types=[
            pltpu.VMEM((CHUNK, D), jnp.float32),
            pltpu.VMEM((CHUNK, D), jnp.float32),
            pltpu.VMEM((CHUNK, 16), jnp.float32),
            pltpu.VMEM((CHUNK, 16), jnp.float32),
            pltpu.VMEM((CHUNK,), jnp.float32),
            pltpu.VMEM((CHUNK,), jnp.float32),
            pltpu.VMEM((1, CHUNK), jnp.int32),
            pltpu.VMEM((1, CHUNK), jnp.int32),
            pltpu.SemaphoreType.DMA,
            pltpu.SemaphoreType.DMA,
            pltpu.VMEM_SHARED((NUM_BAGS, D), jnp.float32),
            pltpu.VMEM_SHARED((NUM_BAGS, 16), jnp.float32),
        ],
        compiler_params=pltpu.CompilerParams(use_tc_tiling_on_sc=False),
    )
    return fn(x, alpha, idx)


def _finish_body(sx_ref, sa_ref, daux_ref, o_ref):
    ssum = sx_ref[...]
    asum = sa_ref[:, 0:1]
    safe = jnp.where(asum != 0.0, asum, 1.0)
    bag = jnp.where(asum != 0.0, ssum / safe, 0.0)
    dw = daux_ref[0:2, :]
    logits = lax.dot_general(bag, dw, (((1,), (1,)), ((), ())),
                             preferred_element_type=jnp.float32)
    logits = logits + daux_ref[2:3, 0:2]
    m = jnp.max(logits, axis=1, keepdims=True)
    e = jnp.exp(logits - m)
    o_ref[...] = e / jnp.sum(e, axis=1, keepdims=True)


def _finish_pass(sx, sa, daux):
    return pl.pallas_call(
        _finish_body,
        out_shape=jax.ShapeDtypeStruct((NUM_BAGS, 2), jnp.float32),
    )(sx, sa, daux)


def kernel(bag_encoding, batch_indices, Vw, Vb, ww, wb, Dw, Db):
    x = bag_encoding
    vbww = (jnp.zeros((H, 8), jnp.float32)
            .at[:, 0].set(Vb).at[:, 1].set(ww[0]))
    alpha = _alpha_pass(x, Vw, vbww)
    sx, sa = _segsum_pass(x, alpha, batch_indices.astype(jnp.int32))
    daux = jnp.zeros((8, D), jnp.float32).at[0:2].set(Dw).at[2, 0:2].set(Db)
    return _finish_pass(sx, sa, daux)


# speedup vs baseline: 7.2394x; 1.2941x over previous
import functools

import jax
import jax.numpy as jnp
from jax import lax
from jax.experimental import pallas as pl
from jax.experimental.pallas import tpu as pltpu
from jax.experimental.pallas import tpu_sc as plsc

N = 320000
D = 128
H = 64
NUM_BAGS = 10000

NC = 2
NS = 16
HBAGS = NUM_BAGS // NC
CHUNK = 64
STRIPE = 1000
NSTRIPERS = HBAGS // STRIPE

BT = 2048


def _alpha_body(x_ref, vw_ref, vbww_ref, o_ref):
    x = x_ref[...]
    vw = vw_ref[...]
    vt = jnp.tanh(
        lax.dot_general(vw, x, (((1,), (1,)), ((), ())),
                        preferred_element_type=jnp.float32)
        + vbww_ref[:, 0:1])
    s = jnp.sum(vt * vbww_ref[:, 1:2], axis=0)
    o_ref[...] = jnp.exp(s)


def _alpha_pass(x, vw, aux):
    grid = pl.cdiv(N, BT)
    return pl.pallas_call(
        _alpha_body,
        grid=(grid,),
        in_specs=[
            pl.BlockSpec((BT, D), lambda i: (i, 0)),
            pl.BlockSpec((H, D), lambda i: (0, 0)),
            pl.BlockSpec((H, 8), lambda i: (0, 0)),
        ],
        out_specs=pl.BlockSpec((BT,), lambda i: (i,)),
        out_shape=jax.ShapeDtypeStruct((N,), jnp.float32),
    )(x, vw, aux)


def _segsum_body(x_hbm, alpha_hbm, idx_hbm, bnd_hbm, outx_hbm, outa_hbm,
                 rows_v0, rows_v1, a16_v0, a16_v1, alpha_v0, alpha_v1,
                 idx_v0, idx_v1, bnd_v, sem0, sem1, accx_s, acca_s):
    c = lax.axis_index("c")
    s = lax.axis_index("s")
    rows_b = (rows_v0, rows_v1)
    a16_b = (a16_v0, a16_v1)
    alpha_b = (alpha_v0, alpha_v1)
    idx_b = (idx_v0, idx_v1)
    sem_b = (sem0, sem1)

    pltpu.sync_copy(bnd_hbm, bnd_v)
    bnd = bnd_v[:]
    rsplit = bnd[0]
    e_up = bnd[1]
    e_dn = bnd[2]
    start_c = jnp.where(c == 0, 0, e_dn)
    end_c = jnp.where(c == 0, e_up, N)
    own_lo = jnp.where(c == 0, 0, rsplit)
    own_hi = jnp.where(c == 0, rsplit, N)
    per_sub = ((lax.div(end_c - start_c + NS - 1, NS) + 7) // 8) * 8
    mystart = start_c + s * per_sub
    myhi = jnp.minimum(jnp.minimum(mystart + per_sub, end_c), own_hi)
    mylo = jnp.maximum(mystart, own_lo)
    nchunks = jnp.maximum(myhi - mystart, 0 * myhi)
    nchunks = (nchunks + CHUNK - 1) // CHUNK

    def _zrow(r, carry):
        for j in range(D // 16):
            rows_v0[r, pl.ds(16 * j, 16)] = jnp.zeros((16,), jnp.float32)
        a16_v0[r, :] = jnp.zeros((16,), jnp.float32)
        return carry

    lax.fori_loop(0, CHUNK, _zrow, 0, unroll=1)
    zbase = s * STRIPE

    @pl.when(s < NSTRIPERS)
    def _zero_stripe():
        for off in range(0, STRIPE, CHUNK):
            ln = min(CHUNK, STRIPE - off)
            pltpu.sync_copy(rows_v0.at[pl.ds(0, ln)],
                            accx_s.at[pl.ds(zbase + off, ln)])
            pltpu.sync_copy(a16_v0.at[pl.ds(0, ln)],
                            acca_s.at[pl.ds(zbase + off, ln)])

    plsc.subcore_barrier()

    def _load_base(t):
        return pl.multiple_of(
            jnp.minimum(mystart + t * CHUNK, N - CHUNK), 8)

    def _start_loads(t, b):
        base = _load_base(t)
        pltpu.async_copy(x_hbm.at[pl.ds(base, CHUNK)], rows_b[b], sem_b[b])
        pltpu.async_copy(alpha_hbm.at[pl.ds(base, CHUNK)], alpha_b[b],
                         sem_b[b])
        pltpu.async_copy(idx_hbm.at[pl.ds(base, CHUNK)], idx_b[b].at[0],
                         sem_b[b])

    def _wait_loads(b):
        pltpu.make_async_copy(x_hbm.at[pl.ds(0, CHUNK)], rows_b[b],
                              sem_b[b]).wait()
        pltpu.make_async_copy(alpha_hbm.at[pl.ds(0, CHUNK)], alpha_b[b],
                              sem_b[b]).wait()
        pltpu.make_async_copy(idx_hbm.at[pl.ds(0, CHUNK)], idx_b[b].at[0],
                              sem_b[b]).wait()

    def _process(t, b):
        rows_v, a16_v, alpha_v, idx_v = (rows_b[b], a16_b[b], alpha_b[b],
                                         idx_b[b])
        base = _load_base(t)
        nom = mystart + t * CHUNK
        lo_t = jnp.maximum(nom, mylo)
        hi_t = jnp.minimum(nom + CHUNK, myhi)
        lanes = lax.iota(jnp.int32, 16)

        def _rowgrp(g, carry2):
            pos = base + 16 * g + lanes
            ok = (pos >= lo_t) & (pos < hi_t)
            av = jnp.where(ok, alpha_v[pl.ds(16 * g, 16)], 0.0)
            iv = idx_v[0, pl.ds(16 * g, 16)] - c * HBAGS
            idx_v[0, pl.ds(16 * g, 16)] = jnp.clip(iv, 0, HBAGS - 1)
            for k in range(16):
                r = 16 * g + k
                a = jnp.full((16,), av[k], jnp.float32)
                for j in range(D // 16):
                    rows_v[r, pl.ds(16 * j, 16)] = (
                        rows_v[r, pl.ds(16 * j, 16)] * a)
                a16_v[r, :] = a
            return carry2

        lax.fori_loop(0, CHUNK // 16, _rowgrp, 0, unroll=1)
        pltpu.sync_copy(rows_v, accx_s.at[idx_v.at[0]], add=True)
        pltpu.sync_copy(a16_v, acca_s.at[idx_v.at[0]], add=True)

    @pl.when(nchunks > 0)
    def _prologue():
        _start_loads(0, 0)

    last = jnp.maximum(nchunks - 1, 0)

    def _pair(u, carry):
        t0 = 2 * u
        _wait_loads(0)
        _start_loads(jnp.minimum(t0 + 1, last), 1)
        _process(t0, 0)
        _wait_loads(1)
        _start_loads(jnp.minimum(t0 + 2, last), 0)

        @pl.when(t0 + 1 < nchunks)
        def _odd():
            _process(t0 + 1, 1)

        return carry

    lax.fori_loop(0, (nchunks + 1) // 2, _pair, 0, unroll=1)

    @pl.when(nchunks > 0)
    def _drain():
        _wait_loads(0)

    plsc.subcore_barrier()

    @pl.when(s < NSTRIPERS)
    def _copy_out():
        pltpu.sync_copy(accx_s.at[pl.ds(zbase, STRIPE)],
                        outx_hbm.at[pl.ds(c * HBAGS + zbase, STRIPE)])
        pltpu.sync_copy(acca_s.at[pl.ds(zbase, STRIPE)],
                        outa_hbm.at[pl.ds(c * HBAGS + zbase, STRIPE)])


def _segsum_pass(x, alpha, idx):
    mesh = plsc.VectorSubcoreMesh(core_axis_name="c", subcore_axis_name="s",
                                  num_cores=NC)
    fn = pl.kernel(
        _segsum_body,
        out_type=[
            jax.ShapeDtypeStruct((NUM_BAGS, D), jnp.float32),
            jax.ShapeDtypeStruct((NUM_BAGS, 16), jnp.float32),
        ],
        mesh=mesh,
        scratch_types=[
            pltpu.VMEM((CHUNK, D), jnp.float32),
            pltpu.VMEM((CHUNK, D), jnp.float32),
            pltpu.VMEM((CHUNK, 16), jnp.float32),
            pltpu.VMEM((CHUNK, 16), jnp.float32),
            pltpu.VMEM((CHUNK,), jnp.float32),
            pltpu.VMEM((CHUNK,), jnp.float32),
            pltpu.VMEM((1, CHUNK), jnp.int32),
            pltpu.VMEM((1, CHUNK), jnp.int32),
            pltpu.VMEM((16,), jnp.int32),
            pltpu.SemaphoreType.DMA,
            pltpu.SemaphoreType.DMA,
            pltpu.VMEM_SHARED((HBAGS, D), jnp.float32),
            pltpu.VMEM_SHARED((HBAGS, 16), jnp.float32),
        ],
        compiler_params=pltpu.CompilerParams(use_tc_tiling_on_sc=False),
    )
    idx32 = idx
    rsplit = jnp.searchsorted(idx32, HBAGS).astype(jnp.int32)
    bnd = jnp.zeros((16,), jnp.int32)
    bnd = bnd.at[0].set(rsplit)
    bnd = bnd.at[1].set(jnp.minimum((rsplit + 7) // 8 * 8, N))
    bnd = bnd.at[2].set(rsplit // 8 * 8)
    return fn(x, alpha, idx32, bnd)


def _finish_body(sx_ref, sa_ref, daux_ref, o_ref):
    ssum = sx_ref[...]
    asum = sa_ref[:, 0:1]
    safe = jnp.where(asum != 0.0, asum, 1.0)
    bag = jnp.where(asum != 0.0, ssum / safe, 0.0)
    dw = daux_ref[0:2, :]
    logits = lax.dot_general(bag, dw, (((1,), (1,)), ((), ())),
                             preferred_element_type=jnp.float32)
    logits = logits + daux_ref[2:3, 0:2]
    m = jnp.max(logits, axis=1, keepdims=True)
    e = jnp.exp(logits - m)
    o_ref[...] = e / jnp.sum(e, axis=1, keepdims=True)


def _finish_pass(sx, sa, daux):
    return pl.pallas_call(
        _finish_body,
        out_shape=jax.ShapeDtypeStruct((NUM_BAGS, 2), jnp.float32),
    )(sx, sa, daux)


def kernel(bag_encoding, batch_indices, Vw, Vb, ww, wb, Dw, Db):
    x = bag_encoding
    vbww = (jnp.zeros((H, 8), jnp.float32)
            .at[:, 0].set(Vb).at[:, 1].set(ww[0]))
    alpha = _alpha_pass(x, Vw, vbww)
    sx, sa = _segsum_pass(x, alpha, batch_indices.astype(jnp.int32))
    daux = jnp.zeros((8, D), jnp.float32).at[0:2].set(Dw).at[2, 0:2].set(Db)
    return _finish_pass(sx, sa, daux)
